# Initial kernel scaffold; baseline (speedup 1.0000x reference)
import jax
import jax.numpy as jnp
from jax.experimental import pallas as pl


def kernel(x, edge_index0, edge_index1, W1a_dst, W1a_src, W1a_upd, W1b_dst, W1b_src, W1b_upd, W2a_dst, W2a_src, W2a_upd, W2b_dst, W2b_src, W2b_upd, bn1_g, bn1_b, bn2_g, bn2_b):
    raise NotImplementedError("write your pallas kernel here")


# speedup vs baseline: 3.0661x; 3.0661x over previous
import functools

import jax
import jax.numpy as jnp
from jax import lax
from jax.experimental import pallas as pl
from jax.experimental.pallas import tpu as pltpu
from jax.experimental.pallas import tpu_sc as plsc

N = 10000
E = 160000
H = 128
NC = 2
NS = 16
CH = 128
NCHUNK = 79
EPT = NCHUNK * CH
PADN = NS * EPT - E
N_PAD = 10240
STRIPE = N_PAD // NS
DUMMY = N
EPW = 2 * 2 * E // (NC * NS)
GCH = 128
NFULL = EPW // GCH
TAIL = EPW - NFULL * GCH

_PIECES = []
_r = 0
while _r < STRIPE:
    _n = min(CH, STRIPE - _r)
    _PIECES.append((_r, _n))
    _r += _n


@functools.lru_cache(maxsize=None)
def _mk_agg(width):
    mesh = plsc.VectorSubcoreMesh(core_axis_name="c", subcore_axis_name="s",
                                  num_cores=NC, num_subcores=NS)

    def body(table, srcs, dsts, z_in, sums_out,
             acc, idxs, idxd, rows0, sem0):
        c = lax.axis_index("c")
        s = lax.axis_index("s")
        off = pl.multiple_of(s * STRIPE, STRIPE)
        rbase = (c * NS + s) * NCHUNK

        pltpu.sync_copy(z_in, rows0)
        for (r, n) in _PIECES:
            pltpu.sync_copy(rows0.at[pl.ds(0, n)],
                            acc.at[pl.ds(off + r, n)])
        plsc.subcore_barrier()

        @pl.loop(0, NCHUNK)
        def _chunk(j):
            pltpu.sync_copy(srcs.at[rbase + j], idxs)
            gather = pltpu.async_copy(table.at[idxs], rows0, sem0)
            pltpu.sync_copy(dsts.at[rbase + j], idxd)
            gather.wait()
            pltpu.sync_copy(rows0, acc.at[idxd], add=True)

        plsc.subcore_barrier()
        obase = pl.multiple_of(c * N_PAD + s * STRIPE, STRIPE)
        for (r, n) in _PIECES:
            pltpu.sync_copy(acc.at[pl.ds(off + r, n)], rows0.at[pl.ds(0, n)])
            pltpu.sync_copy(rows0.at[pl.ds(0, n)],
                            sums_out.at[pl.ds(obase + r, n)])

    return pl.kernel(
        body,
        out_type=jax.ShapeDtypeStruct((NC * N_PAD, width), jnp.float32),
        mesh=mesh,
        scratch_types=[
            pltpu.VMEM_SHARED((N_PAD, width), jnp.float32),
            pltpu.VMEM((CH,), jnp.int32),
            pltpu.VMEM((CH,), jnp.int32),
            pltpu.VMEM((CH, width), jnp.float32),
            pltpu.SemaphoreType.DMA,
        ])


@functools.lru_cache(maxsize=None)
def _mk_cnt():
    mesh = plsc.VectorSubcoreMesh(core_axis_name="c", subcore_axis_name="s",
                                  num_cores=NC, num_subcores=NS)

    def body(dsts, z_in, ones_in, cnts_out, acc, idxd, rows, sem):
        c = lax.axis_index("c")
        s = lax.axis_index("s")
        off = pl.multiple_of(s * STRIPE, STRIPE)
        rbase = (c * NS + s) * NCHUNK
        pltpu.sync_copy(z_in, rows)
        for (r, n) in _PIECES:
            pltpu.sync_copy(rows.at[pl.ds(0, n)], acc.at[pl.ds(off + r, n)])
        pltpu.sync_copy(ones_in, rows)
        plsc.subcore_barrier()

        @pl.loop(0, NCHUNK)
        def _chunk(j):
            pltpu.sync_copy(dsts.at[rbase + j], idxd)
            pltpu.sync_copy(rows, acc.at[idxd], add=True)

        plsc.subcore_barrier()
        obase = pl.multiple_of(c * N_PAD + s * STRIPE, STRIPE)
        for (r, n) in _PIECES:
            pltpu.sync_copy(acc.at[pl.ds(off + r, n)], rows.at[pl.ds(0, n)])
            pltpu.sync_copy(rows.at[pl.ds(0, n)],
                            cnts_out.at[pl.ds(obase + r, n)])

    return pl.kernel(
        body,
        out_type=jax.ShapeDtypeStruct((NC * N_PAD, H), jnp.float32),
        mesh=mesh,
        scratch_types=[
            pltpu.VMEM_SHARED((N_PAD, H), jnp.float32),
            pltpu.VMEM((CH,), jnp.int32),
            pltpu.VMEM((CH, H), jnp.float32),
            pltpu.SemaphoreType.DMA,
        ])


def _edge_gather_body(table, bigidx, out, idx_v, rows0, rows1, rows_t,
                      sem0, sem1):
    c = lax.axis_index("c")
    s = lax.axis_index("s")
    wid = s * NC + c
    base = pl.multiple_of(wid * EPW, EPW)
    pltpu.sync_copy(bigidx.at[pl.ds(base, EPW)], idx_v)
    NB = 13
    NGRP = NFULL // NB
    bufs = (rows0, rows1)
    sems = (sem0, sem1)

    def g_src(o):
        return table.at[idx_v.at[pl.ds(o, GCH)]]

    @pl.loop(0, NGRP)
    def _grp(g):
        gb = pl.multiple_of(g * NB * GCH, GCH)
        pltpu.async_copy(g_src(gb), bufs[0], sems[0])
        for jj in range(NB):
            o = pl.multiple_of(gb + jj * GCH, GCH)
            if jj + 1 < NB:
                o1 = pl.multiple_of(gb + (jj + 1) * GCH, GCH)
                pltpu.async_copy(g_src(o1), bufs[(jj + 1) % 2],
                                 sems[(jj + 1) % 2])
            pltpu.make_async_copy(g_src(o), bufs[jj % 2], sems[jj % 2]).wait()
            pltpu.sync_copy(bufs[jj % 2], out.at[pl.ds(base + o, GCH)])

    pltpu.async_copy(table.at[idx_v.at[pl.ds(NFULL * GCH, TAIL)]],
                     rows_t, sem0).wait()
    pltpu.sync_copy(rows_t, out.at[pl.ds(base + NFULL * GCH, TAIL)])


@functools.lru_cache(maxsize=None)
def _mk_edge_gather():
    return pl.kernel(
        _edge_gather_body,
        out_type=jax.ShapeDtypeStruct((2 * 2 * E, H), jnp.float32),
        mesh=plsc.VectorSubcoreMesh(core_axis_name="c", subcore_axis_name="s",
                                    num_cores=NC, num_subcores=NS),
        scratch_types=[
            pltpu.VMEM((EPW,), jnp.int32),
            pltpu.VMEM((GCH, H), jnp.float32),
            pltpu.VMEM((GCH, H), jnp.float32),
            pltpu.VMEM((TAIL, H), jnp.float32),
            pltpu.SemaphoreType.DMA,
            pltpu.SemaphoreType.DMA,
        ])


def _mmt(a, w):
    return lax.dot_general(a, w, (((1,), (1,)), ((), ())),
                           preferred_element_type=jnp.float32)


def _dense_body(t_ref, ps_ref, cs_ref,
                wda_ref, wsa_ref, wua_ref, wdb_ref, wsb_ref, wub_ref,
                g_ref, b_ref, o_ref):
    T = t_ref[...]
    ps = ps_ref[...]
    cs = cs_ref[...]

    def rel(p, cnt, wd_ref, ws_ref, wu_ref):
        m = p / jnp.maximum(cnt, 1.0)
        wu = wu_ref[...]
        a = _mmt(T, wd_ref[...])
        b = _mmt(m, ws_ref[...])
        return _mmt(a, wu[:, :H]) + _mmt(b, wu[:, H:])

    pre = 0.5 * (rel(ps[0:N], cs[0:N, :1], wda_ref, wsa_ref, wua_ref)
                 + rel(ps[N_PAD:N_PAD + N], cs[N_PAD:N_PAD + N, :1],
                       wdb_ref, wsb_ref, wub_ref))
    mu = jnp.mean(pre, axis=0, keepdims=True)
    var = jnp.mean((pre - mu) ** 2, axis=0, keepdims=True)
    y = g_ref[...] * (pre - mu) * lax.rsqrt(var + 1.0) + b_ref[...]
    o_ref[...] = jnp.where(y > 0, y, 0.01 * y)


_dense = pl.pallas_call(
    _dense_body,
    out_shape=jax.ShapeDtypeStruct((N, H), jnp.float32))


def _pad_flat(v, fill):
    return jnp.concatenate([v, jnp.full((PADN,), fill, jnp.int32)])


def kernel(x, edge_index0, edge_index1,
           W1a_dst, W1a_src, W1a_upd, W1b_dst, W1b_src, W1b_upd,
           W2a_dst, W2a_src, W2a_upd, W2b_dst, W2b_src, W2b_upd,
           bn1_g, bn1_b, bn2_g, bn2_b):
    srcs = jnp.concatenate([_pad_flat(edge_index0[0], 0),
                            _pad_flat(edge_index1[0], 0)])
    dsts = jnp.concatenate([_pad_flat(edge_index0[1], DUMMY),
                            _pad_flat(edge_index1[1], DUMMY)])
    srcs = srcs.reshape(-1, CH)
    dsts = dsts.reshape(-1, CH)
    z_in = jnp.zeros((CH, H), jnp.float32)
    ones_in = jnp.ones((CH, H), jnp.float32)
    cnts16 = _mk_cnt()(dsts, z_in, ones_in)[:, :16]
    sums1 = _mk_agg(H)(x, srcs, dsts, z_in)
    h1 = _dense(x, sums1, cnts16,
                W1a_dst, W1a_src, W1a_upd, W1b_dst, W1b_src, W1b_upd,
                bn1_g.reshape(1, H), bn1_b.reshape(1, H))
    sums2 = _mk_agg(H)(h1, srcs, dsts, z_in)
    h2 = _dense(h1, sums2, cnts16,
                W2a_dst, W2a_src, W2a_upd, W2b_dst, W2b_src, W2b_upd,
                bn2_g.reshape(1, H), bn2_b.reshape(1, H))

    bigidx = jnp.concatenate([edge_index0.T.reshape(-1),
                              edge_index1.T.reshape(-1)])
    ef = _mk_edge_gather()(h2, bigidx)
    return ef.reshape(2 * E, 2 * H)
